# baseline (speedup 1.0000x reference)
import dataclasses
import functools

import jax
import jax.numpy as jnp
from jax import lax
from jax.experimental import pallas as pl
from jax.experimental.pallas import tpu as pltpu
from jax.experimental.pallas import tpu_sc as plsc

N_NODES = 10000
D = 128
REL_PAD = 512
NC = 2
NS = 16
NW = NC * NS
C_SZ = 40


def _vmesh():
    return plsc.VectorSubcoreMesh(core_axis_name="c", subcore_axis_name="s")


def _sc_params():
    cp = pltpu.CompilerParams()
    if "needs_layout_passes" in pltpu.CompilerParams.__dataclass_fields__:
        cp = dataclasses.replace(cp, needs_layout_passes=False)
    return cp


def _sc_gather(table, idx):
    b = idx.shape[0]
    ch = 40
    n_ch = b // ch
    n_loop = (n_ch + NW - 1) // NW

    @functools.partial(
        pl.kernel,
        out_type=jax.ShapeDtypeStruct((b, D), jnp.float32),
        mesh=_vmesh(),
        scratch_types=[
            pltpu.VMEM((ch,), jnp.int32),
            pltpu.VMEM((ch, D), jnp.float32),
            pltpu.SemaphoreType.DMA,
        ],
    )
    def k(table_hbm, idx_hbm, out_hbm, idx_v, rows_v, sem):
        w = lax.axis_index("c") * NS + lax.axis_index("s")

        @pl.loop(0, n_loop)
        def _(j):
            cidx = j * NW + w

            @pl.when(cidx < n_ch)
            def _():
                base = cidx * ch
                pltpu.sync_copy(idx_hbm.at[pl.ds(base, ch)], idx_v)
                pltpu.async_copy(table_hbm.at[idx_v], rows_v, sem).wait()
                pltpu.sync_copy(rows_v, out_hbm.at[pl.ds(base, ch)])

    return k(table, idx)


def _sc_edge(hw, rp, sa4, dst3):
    n_chunk = sa4.shape[1]
    n_half = n_chunk // 2
    z_ch = C_SZ
    n_zch = N_NODES // z_ch
    n_zloop = (n_zch + NS - 1) // NS

    @functools.partial(
        pl.kernel,
        out_type=jax.ShapeDtypeStruct((NC, N_NODES, D), jnp.float32),
        mesh=_vmesh(),
        compiler_params=_sc_params(),
        scratch_types=[
            pltpu.VMEM_SHARED((N_NODES, D), jnp.float32),
            pltpu.VMEM((C_SZ,), jnp.int32),
            pltpu.VMEM((C_SZ,), jnp.int32),
            pltpu.VMEM((2, C_SZ), jnp.int32),
            pltpu.VMEM((2, C_SZ), jnp.int32),
            pltpu.VMEM((C_SZ, D), jnp.float32),
            pltpu.VMEM((C_SZ, D), jnp.float32),
            pltpu.VMEM((C_SZ, 2 * D), jnp.float32),
            pltpu.VMEM((C_SZ, 2 * D), jnp.float32),
            pltpu.SemaphoreType.DMA,
            pltpu.SemaphoreType.DMA,
            pltpu.SemaphoreType.DMA,
            pltpu.SemaphoreType.DMA,
            pltpu.SemaphoreType.DMA,
            pltpu.SemaphoreType.DMA,
            pltpu.SemaphoreType.DMA,
            pltpu.SemaphoreType.DMA,
            pltpu.SemaphoreType.DMA,
            pltpu.SemaphoreType.DMA,
        ],
    )
    def k(hw_hbm, rp_hbm, sa_hbm, dst_hbm, out_hbm,
          agg_sh, d_a, d_b, i_a, i_b, h_a, h_b, e_a, e_b,
          sd_a, sd_b, si_a, si_b, sh_a, sh_b, se_a, se_b, ss_a, ss_b):
        c = lax.axis_index("c")
        s = lax.axis_index("s")
        g = c * NS + s
        d_v = (d_a, d_b)
        i_v = (i_a, i_b)
        h_v = (h_a, h_b)
        e_v = (e_a, e_b)
        sd = (sd_a, sd_b)
        si = (si_a, si_b)
        sh = (sh_a, sh_b)
        se = (se_a, se_b)
        ss = (ss_a, ss_b)

        @pl.loop(0, C_SZ)
        def _(r):
            for cc in range(D // 16):
                h_a[r, pl.ds(cc * 16, 16)] = jnp.zeros((16,), jnp.float32)

        @pl.loop(0, n_zloop)
        def _(j):
            zc = j * NS + s

            @pl.when(zc < n_zch)
            def _():
                pltpu.sync_copy(h_a.at[pl.ds(0, z_ch)],
                                agg_sh.at[pl.ds(zc * z_ch, z_ch)])

        plsc.subcore_barrier()

        def idx_start(i, b):
            pltpu.async_copy(sa_hbm.at[g].at[i], i_v[b], si[b])

        def idx_wait(b):
            pltpu.make_async_copy(sa_hbm.at[g].at[0], i_v[b], si[b]).wait()

        def dst_start(i, b):
            pltpu.async_copy(dst_hbm.at[g].at[i], d_v[b], sd[b])

        def dst_wait(b):
            pltpu.make_async_copy(dst_hbm.at[g].at[0], d_v[b], sd[b]).wait()

        def gather_start(b):
            pltpu.async_copy(hw_hbm.at[i_v[b].at[0]], h_v[b], sh[b])
            pltpu.async_copy(rp_hbm.at[i_v[b].at[1]], e_v[b], se[b])

        def gather_wait(b):
            pltpu.make_async_copy(hw_hbm.at[i_v[b].at[0]], h_v[b], sh[b]).wait()
            pltpu.make_async_copy(rp_hbm.at[i_v[b].at[1]], e_v[b], se[b]).wait()

        def compute(b):
            hb, eb = h_v[b], e_v[b]

            @pl.loop(0, C_SZ)
            def _(r):
                pe = eb[r, pl.ds(D, 16)]
                for cc in range(D // 16):
                    sl = pl.ds(cc * 16, 16)
                    hb[r, sl] = jnp.maximum(hb[r, sl] + eb[r, sl], 0.0) * pe

        def scat_start(b):
            pltpu.async_copy(h_v[b], agg_sh.at[d_v[b]], ss[b], add=True)

        def scat_wait(b):
            pltpu.make_async_copy(h_v[b], agg_sh.at[d_v[b]], ss[b]).wait()

        idx_start(0, 0)
        idx_start(1, 1)
        dst_start(0, 0)
        dst_start(1, 1)
        idx_wait(0)
        gather_start(0)

        @pl.loop(0, n_half)
        def _(j):
            a = 2 * j

            @pl.when(j > 0)
            def _():
                scat_wait(1)
                dst_start(a + 1, 1)

            idx_wait(1)
            gather_start(1)
            gather_wait(0)
            compute(0)

            @pl.when(a + 2 < n_chunk)
            def _():
                idx_start(a + 2, 0)

            dst_wait(0)
            scat_start(0)

            scat_wait(0)

            @pl.when(a + 2 < n_chunk)
            def _():
                dst_start(a + 2, 0)
                idx_wait(0)
                gather_start(0)

            gather_wait(1)
            compute(1)

            @pl.when(a + 3 < n_chunk)
            def _():
                idx_start(a + 3, 1)

            dst_wait(1)
            scat_start(1)

        scat_wait(1)

        plsc.subcore_barrier()

        @pl.loop(0, n_zloop)
        def _(j):
            zc = j * NS + s

            @pl.when(zc < n_zch)
            def _():
                sl = pl.ds(zc * z_ch, z_ch)
                pltpu.sync_copy(agg_sh.at[sl], out_hbm.at[c].at[sl])

    return k(hw, rp, sa4, dst3)


def _tc_prep(h, rel_p, prior_f, wm):
    def body(h_ref, rel_ref, p_ref, w_ref, hw_ref, rp_ref):
        w = w_ref[...]
        hw_ref[...] = jnp.dot(h_ref[...], w, preferred_element_type=jnp.float32)
        rp_ref[:, :D] = jnp.dot(rel_ref[...], w, preferred_element_type=jnp.float32)
        rp_ref[:, D:] = jnp.broadcast_to(p_ref[...][:, None], (REL_PAD, D))

    return pl.pallas_call(
        body,
        out_shape=(
            jax.ShapeDtypeStruct((N_NODES, D), jnp.float32),
            jax.ShapeDtypeStruct((REL_PAD, 2 * D), jnp.float32),
        ),
    )(h, rel_p, prior_f, wm)


def _tc_update(h, agg2, ws, wa):
    def body(h_ref, a_ref, ws_ref, wa_ref, o_ref):
        agg = a_ref[0] + a_ref[1]
        o_ref[...] = jnp.maximum(
            jnp.dot(h_ref[...], ws_ref[...], preferred_element_type=jnp.float32)
            + jnp.dot(agg, wa_ref[...], preferred_element_type=jnp.float32),
            0.0,
        )

    return pl.pallas_call(
        body,
        out_shape=jax.ShapeDtypeStruct((N_NODES, D), jnp.float32),
    )(h, agg2, ws, wa)


def kernel(x, edge_index, edge_attr, entity_emb, relation_emb, relation_prior,
           W_msg1, W_self1, W_agg1, W_msg2, W_self2, W_agg2):
    x = x.astype(jnp.int32)
    e_total = edge_index.shape[1]
    ept = e_total // NW
    n_chunk = ept // C_SZ
    src3 = edge_index[0].astype(jnp.int32).reshape(NW, n_chunk, 1, C_SZ)
    attr3 = edge_attr.astype(jnp.int32).reshape(NW, n_chunk, 1, C_SZ)
    sa4 = jnp.concatenate([src3, attr3], axis=2)
    dst3 = edge_index[1].astype(jnp.int32).reshape(NW, n_chunk, C_SZ)
    nrel = relation_emb.shape[0]
    rel_p = jnp.zeros((REL_PAD, D), jnp.float32).at[:nrel].set(relation_emb)
    prior_f = jnp.zeros((REL_PAD,), jnp.float32).at[:nrel].set(
        relation_prior.reshape(-1))

    h = _sc_gather(entity_emb, x)
    for wm, ws, wa in ((W_msg1, W_self1, W_agg1), (W_msg2, W_self2, W_agg2)):
        hw, rp = _tc_prep(h, rel_p, prior_f, wm)
        agg2 = _sc_edge(hw, rp, sa4, dst3)
        h = _tc_update(h, agg2, ws, wa)
    return h


# speedup vs baseline: 1.1245x; 1.1245x over previous
import dataclasses
import functools

import jax
import jax.numpy as jnp
from jax import lax
from jax.experimental import pallas as pl
from jax.experimental.pallas import tpu as pltpu
from jax.experimental.pallas import tpu_sc as plsc

N_NODES = 10000
D = 128
REL_PAD = 512
NC = 2
NS = 16
NW = NC * NS
C_SZ = 40


def _vmesh():
    return plsc.VectorSubcoreMesh(core_axis_name="c", subcore_axis_name="s")


def _sc_params():
    cp = pltpu.CompilerParams()
    if "needs_layout_passes" in pltpu.CompilerParams.__dataclass_fields__:
        cp = dataclasses.replace(cp, needs_layout_passes=False)
    return cp


def _sc_gather(table, idx):
    b = idx.shape[0]
    ch = 40
    n_ch = b // ch
    n_loop = (n_ch + NW - 1) // NW

    @functools.partial(
        pl.kernel,
        out_type=jax.ShapeDtypeStruct((b, D), jnp.float32),
        mesh=_vmesh(),
        scratch_types=[
            pltpu.VMEM((ch,), jnp.int32),
            pltpu.VMEM((ch, D), jnp.float32),
            pltpu.SemaphoreType.DMA,
        ],
    )
    def k(table_hbm, idx_hbm, out_hbm, idx_v, rows_v, sem):
        w = lax.axis_index("c") * NS + lax.axis_index("s")

        @pl.loop(0, n_loop)
        def _(j):
            cidx = j * NW + w

            @pl.when(cidx < n_ch)
            def _():
                base = cidx * ch
                pltpu.sync_copy(idx_hbm.at[pl.ds(base, ch)], idx_v)
                pltpu.async_copy(table_hbm.at[idx_v], rows_v, sem).wait()
                pltpu.sync_copy(rows_v, out_hbm.at[pl.ds(base, ch)])

    return k(table, idx)


def _sc_edge(hw, rp, sa4, dst3):
    n_chunk = sa4.shape[1]
    n_half = n_chunk // 2
    z_ch = C_SZ
    n_zch = N_NODES // z_ch
    n_zloop = (n_zch + NS - 1) // NS

    @functools.partial(
        pl.kernel,
        out_type=jax.ShapeDtypeStruct((NC, N_NODES, D), jnp.float32),
        mesh=_vmesh(),
        compiler_params=_sc_params(),
        scratch_types=[
            pltpu.VMEM_SHARED((N_NODES, D), jnp.float32),
            pltpu.VMEM((C_SZ,), jnp.int32),
            pltpu.VMEM((C_SZ,), jnp.int32),
            pltpu.VMEM((2, C_SZ), jnp.int32),
            pltpu.VMEM((2, C_SZ), jnp.int32),
            pltpu.VMEM((C_SZ, D), jnp.float32),
            pltpu.VMEM((C_SZ, D), jnp.float32),
            pltpu.VMEM((C_SZ, 2 * D), jnp.float32),
            pltpu.VMEM((C_SZ, 2 * D), jnp.float32),
            pltpu.VMEM((C_SZ, D), jnp.float32),
            pltpu.VMEM((C_SZ, D), jnp.float32),
            pltpu.SemaphoreType.DMA,
            pltpu.SemaphoreType.DMA,
            pltpu.SemaphoreType.DMA,
            pltpu.SemaphoreType.DMA,
            pltpu.SemaphoreType.DMA,
            pltpu.SemaphoreType.DMA,
            pltpu.SemaphoreType.DMA,
            pltpu.SemaphoreType.DMA,
            pltpu.SemaphoreType.DMA,
            pltpu.SemaphoreType.DMA,
        ],
    )
    def k(hw_hbm, rp_hbm, sa_hbm, dst_hbm, out_hbm,
          agg_sh, d_a, d_b, i_a, i_b, h_a, h_b, e_a, e_b, o_a, o_b,
          sd_a, sd_b, si_a, si_b, sh_a, sh_b, se_a, se_b, ss_a, ss_b):
        c = lax.axis_index("c")
        s = lax.axis_index("s")
        g = c * NS + s
        d_v = (d_a, d_b)
        i_v = (i_a, i_b)
        h_v = (h_a, h_b)
        e_v = (e_a, e_b)
        o_v = (o_a, o_b)
        sd = (sd_a, sd_b)
        si = (si_a, si_b)
        sh = (sh_a, sh_b)
        se = (se_a, se_b)
        ss = (ss_a, ss_b)

        @pl.loop(0, C_SZ)
        def _(r):
            for cc in range(D // 16):
                h_a[r, pl.ds(cc * 16, 16)] = jnp.zeros((16,), jnp.float32)

        @pl.loop(0, n_zloop)
        def _(j):
            zc = j * NS + s

            @pl.when(zc < n_zch)
            def _():
                pltpu.sync_copy(h_a.at[pl.ds(0, z_ch)],
                                agg_sh.at[pl.ds(zc * z_ch, z_ch)])

        plsc.subcore_barrier()

        def idx_start(i, b):
            pltpu.async_copy(sa_hbm.at[g].at[i], i_v[b], si[b])

        def idx_wait(b):
            pltpu.make_async_copy(sa_hbm.at[g].at[0], i_v[b], si[b]).wait()

        def dst_start(i, b):
            pltpu.async_copy(dst_hbm.at[g].at[i], d_v[b], sd[b])

        def dst_wait(b):
            pltpu.make_async_copy(dst_hbm.at[g].at[0], d_v[b], sd[b]).wait()

        def gather_start(b):
            pltpu.async_copy(hw_hbm.at[i_v[b].at[0]], h_v[b], sh[b])
            pltpu.async_copy(rp_hbm.at[i_v[b].at[1]], e_v[b], se[b])

        def gather_wait(b):
            pltpu.make_async_copy(hw_hbm.at[i_v[b].at[0]], h_v[b], sh[b]).wait()
            pltpu.make_async_copy(rp_hbm.at[i_v[b].at[1]], e_v[b], se[b]).wait()

        def compute(b):
            hb, eb, ob = h_v[b], e_v[b], o_v[b]

            @pl.loop(0, C_SZ)
            def _(r):
                pe = eb[r, pl.ds(D, 16)]
                for cc in range(D // 16):
                    sl = pl.ds(cc * 16, 16)
                    ob[r, sl] = jnp.maximum(hb[r, sl] + eb[r, sl], 0.0) * pe

        def scat_start(b):
            pltpu.async_copy(o_v[b], agg_sh.at[d_v[b]], ss[b], add=True)

        def scat_wait(b):
            pltpu.make_async_copy(o_v[b], agg_sh.at[d_v[b]], ss[b]).wait()

        idx_start(0, 0)
        idx_start(1, 1)
        dst_start(0, 0)
        dst_start(1, 1)
        idx_wait(0)
        gather_start(0)
        idx_wait(1)
        gather_start(1)

        @pl.loop(0, n_half)
        def _(j):
            a = 2 * j

            gather_wait(0)

            @pl.when(a + 2 < n_chunk)
            def _():
                idx_start(a + 2, 0)

            @pl.when(j > 0)
            def _():
                scat_wait(0)
                dst_start(a, 0)

            compute(0)

            @pl.when(a + 2 < n_chunk)
            def _():
                idx_wait(0)
                gather_start(0)

            dst_wait(0)
            scat_start(0)

            gather_wait(1)

            @pl.when(a + 3 < n_chunk)
            def _():
                idx_start(a + 3, 1)

            @pl.when(j > 0)
            def _():
                scat_wait(1)
                dst_start(a + 1, 1)

            compute(1)

            @pl.when(a + 3 < n_chunk)
            def _():
                idx_wait(1)
                gather_start(1)

            dst_wait(1)
            scat_start(1)

        scat_wait(0)
        scat_wait(1)

        plsc.subcore_barrier()

        @pl.loop(0, n_zloop)
        def _(j):
            zc = j * NS + s

            @pl.when(zc < n_zch)
            def _():
                sl = pl.ds(zc * z_ch, z_ch)
                pltpu.sync_copy(agg_sh.at[sl], out_hbm.at[c].at[sl])

    return k(hw, rp, sa4, dst3)


def _tc_prep(h, rel_p, prior_f, wm):
    def body(h_ref, rel_ref, p_ref, w_ref, hw_ref, rp_ref):
        w = w_ref[...]
        hw_ref[...] = jnp.dot(h_ref[...], w, preferred_element_type=jnp.float32)
        rp_ref[:, :D] = jnp.dot(rel_ref[...], w, preferred_element_type=jnp.float32)
        rp_ref[:, D:] = jnp.broadcast_to(p_ref[...][:, None], (REL_PAD, D))

    return pl.pallas_call(
        body,
        out_shape=(
            jax.ShapeDtypeStruct((N_NODES, D), jnp.float32),
            jax.ShapeDtypeStruct((REL_PAD, 2 * D), jnp.float32),
        ),
    )(h, rel_p, prior_f, wm)


def _tc_update(h, agg2, ws, wa):
    def body(h_ref, a_ref, ws_ref, wa_ref, o_ref):
        agg = a_ref[0] + a_ref[1]
        o_ref[...] = jnp.maximum(
            jnp.dot(h_ref[...], ws_ref[...], preferred_element_type=jnp.float32)
            + jnp.dot(agg, wa_ref[...], preferred_element_type=jnp.float32),
            0.0,
        )

    return pl.pallas_call(
        body,
        out_shape=jax.ShapeDtypeStruct((N_NODES, D), jnp.float32),
    )(h, agg2, ws, wa)


def kernel(x, edge_index, edge_attr, entity_emb, relation_emb, relation_prior,
           W_msg1, W_self1, W_agg1, W_msg2, W_self2, W_agg2):
    x = x.astype(jnp.int32)
    e_total = edge_index.shape[1]
    ept = e_total // NW
    n_chunk = ept // C_SZ
    src3 = edge_index[0].astype(jnp.int32).reshape(NW, n_chunk, 1, C_SZ)
    attr3 = edge_attr.astype(jnp.int32).reshape(NW, n_chunk, 1, C_SZ)
    sa4 = jnp.concatenate([src3, attr3], axis=2)
    dst3 = edge_index[1].astype(jnp.int32).reshape(NW, n_chunk, C_SZ)
    nrel = relation_emb.shape[0]
    rel_p = jnp.zeros((REL_PAD, D), jnp.float32).at[:nrel].set(relation_emb)
    prior_f = jnp.zeros((REL_PAD,), jnp.float32).at[:nrel].set(
        relation_prior.reshape(-1))

    h = _sc_gather(entity_emb, x)
    for wm, ws, wa in ((W_msg1, W_self1, W_agg1), (W_msg2, W_self2, W_agg2)):
        hw, rp = _tc_prep(h, rel_p, prior_f, wm)
        agg2 = _sc_edge(hw, rp, sa4, dst3)
        h = _tc_update(h, agg2, ws, wa)
    return h
